# baseline (speedup 1.0000x reference)
import functools

import jax
import jax.numpy as jnp
from jax.experimental import pallas as pl
from jax.experimental.pallas import tpu as pltpu


def _feat_body(x_ref, w1_ref, b1_ref, w2_ref, b2_ref, mask_ref, pool_ref,
               out_ref, xsrc_ref, stk_ref, y1s_ref, y2_ref, *,
               wimg, n_chunks):
    nb, cin, P0 = x_ref.shape
    cmid = w1_ref.shape[0]
    L = y2_ref.shape[1]
    CH = L // n_chunks
    E = (xsrc_ref.shape[1] - L) // 2
    shifts = [dy * wimg + dx for dy in (-1, 0, 1) for dx in (-1, 0, 1)]

    xsrc_ref[:, 0:E] = jnp.zeros((cin, E), jnp.bfloat16)
    xsrc_ref[:, E + L:] = jnp.zeros((cin, E), jnp.bfloat16)
    for k in range(nb):
        xsrc_ref[:, E + k * P0:E + (k + 1) * P0] = x_ref[k].astype(
            jnp.bfloat16)

    for c in range(n_chunks):
        lo = c * CH
        buf = stk_ref.at[c % 2]
        for t, s in enumerate(shifts):
            buf[t * cin:(t + 1) * cin, :] = (
                xsrc_ref[:, E + s + lo:E + s + lo + CH]
                * mask_ref[t:t + 1, lo:lo + CH])
        acc1 = jnp.dot(w1_ref[...], buf[...],
                       preferred_element_type=jnp.float32)
        y1s_ref[:, E + lo:E + lo + CH] = jnp.maximum(
            acc1 + b1_ref[...], 0.0).astype(jnp.bfloat16)
    y1s_ref[:, 0:E] = jnp.zeros((cmid, E), jnp.bfloat16)
    y1s_ref[:, E + L:] = jnp.zeros((cmid, E), jnp.bfloat16)

    for c in range(n_chunks):
        lo = c * CH
        buf = stk_ref.at[c % 2]
        for t, s in enumerate(shifts):
            buf[t * cmid:(t + 1) * cmid, :] = (
                y1s_ref[:, E + s + lo:E + s + lo + CH]
                * mask_ref[t:t + 1, lo:lo + CH])
        acc2 = jnp.dot(w2_ref[...], buf[...],
                       preferred_element_type=jnp.float32)
        y2_ref[:, lo:lo + CH] = jnp.maximum(
            acc2 + b2_ref[...], 0.0).astype(jnp.bfloat16)

    feats = jnp.dot(y2_ref[...], pool_ref[...],
                    preferred_element_type=jnp.float32)
    out_ref[...] = feats.T


def _extract_features(x_flat, w1, b1, w2, b2, H, W, images_per_block):
    BS, Cin, P0 = x_flat.shape
    Cmid = w1.shape[0]
    Chid = w2.shape[0]
    nb = images_per_block
    nblk = BS // nb
    L = nb * P0
    E = W + 1
    n_chunks = 4

    rr = jnp.arange(H)
    cc = jnp.arange(W)
    tap_masks = []
    for dy in (-1, 0, 1):
        for dx in (-1, 0, 1):
            rv = (rr + dy >= 0) & (rr + dy < H)
            cv = (cc + dx >= 0) & (cc + dx < W)
            tap_masks.append(jnp.tile((rv[:, None] & cv[None, :])
                                      .reshape(P0), (nb,)))
    masks = jnp.stack(tap_masks).astype(jnp.bfloat16)
    pool = jnp.kron(jnp.eye(nb, dtype=jnp.float32),
                    jnp.full((P0, 1), 1.0 / P0)).astype(jnp.bfloat16)

    body = functools.partial(_feat_body, wimg=W, n_chunks=n_chunks)
    return pl.pallas_call(
        body,
        out_shape=jax.ShapeDtypeStruct((nb, nblk * Chid), jnp.float32),
        grid_spec=pltpu.PrefetchScalarGridSpec(
            num_scalar_prefetch=0,
            grid=(nblk,),
            in_specs=[
                pl.BlockSpec((nb, Cin, P0), lambda j: (j, 0, 0)),
                pl.BlockSpec((Cmid, 9 * Cin), lambda j: (0, 0)),
                pl.BlockSpec((Cmid, 1), lambda j: (0, 0)),
                pl.BlockSpec((Chid, 9 * Cmid), lambda j: (0, 0)),
                pl.BlockSpec((Chid, 1), lambda j: (0, 0)),
                pl.BlockSpec((9, L), lambda j: (0, 0)),
                pl.BlockSpec((L, nb), lambda j: (0, 0)),
            ],
            out_specs=pl.BlockSpec((nb, Chid), lambda j: (0, j)),
            scratch_shapes=[
                pltpu.VMEM((Cin, 2 * E + L), jnp.bfloat16),
                pltpu.VMEM((2, 9 * Cin, L // n_chunks), jnp.bfloat16),
                pltpu.VMEM((Cmid, 2 * E + L), jnp.bfloat16),
                pltpu.VMEM((Chid, L), jnp.bfloat16),
            ]),
        compiler_params=pltpu.CompilerParams(
            dimension_semantics=("arbitrary",)),
    )(x_flat, w1, b1, w2, b2, masks, pool)


def _gru_body(feats_ref, wih_ref, whh_ref, bih_ref, bhh_ref, wfc_ref, bfc_ref,
              perm_ref, out1_ref, out2_ref, hs_ref, *,
              batch, seq_len, future_steps):
    Hd = whh_ref.shape[0]
    B, S, F = batch, seq_len, future_steps
    n1 = out1_ref.shape[0]

    wih = wih_ref[...]
    whh = whh_ref[...]
    bih = bih_ref[...]
    bhh = bhh_ref[...]

    gi_all = jnp.dot(feats_ref[...].astype(jnp.bfloat16), wih,
                     preferred_element_type=jnp.float32) + bih

    def cell(gi, gh, h_prev):
        r = jax.nn.sigmoid(gi[:, :Hd] + gh[:, :Hd])
        z = jax.nn.sigmoid(gi[:, Hd:2 * Hd] + gh[:, Hd:2 * Hd])
        n = jnp.tanh(gi[:, 2 * Hd:] + r * gh[:, 2 * Hd:])
        return (1.0 - z) * n + z * h_prev

    h = jnp.zeros((B, Hd), jnp.float32)
    for t in range(S):
        gh = jnp.dot(h.astype(jnp.bfloat16), whh,
                     preferred_element_type=jnp.float32) + bhh
        h = cell(gi_all[t * B:(t + 1) * B, :], gh, h)
        hs_ref[t * B:(t + 1) * B, :] = h

    zero_h = jnp.zeros((B, Hd), jnp.float32)
    lh = h
    for j in range(F):
        gi = jnp.dot(lh.astype(jnp.bfloat16), wih,
                     preferred_element_type=jnp.float32) + bih
        lh = cell(gi, bhh, zero_h)
        hs_ref[(S + j) * B:(S + j + 1) * B, :] = lh

    hsp = jnp.dot(perm_ref[...], hs_ref[...].astype(jnp.bfloat16),
                  preferred_element_type=jnp.float32)
    fc = (jnp.dot(hsp.astype(jnp.bfloat16), wfc_ref[...],
                  preferred_element_type=jnp.float32) + bfc_ref[...])
    out1_ref[...] = fc[:n1]
    out2_ref[...] = fc[n1:]


def _gru_fc(feats_tb, w_ih, w_hh, b_ih, b_hh, w_fc, b_fc, *,
            batch, seq_len, future_steps, current_index):
    Hd = w_hh.shape[0]
    N = w_fc.shape[1]
    B, S, F = batch, seq_len, future_steps
    T = S + F
    Fp1 = F + 1

    rows = []
    for b in range(B):
        for t in range(T):
            rows.append(t * B + b)
    for b in range(B):
        for q in range(Fp1):
            rows.append((current_index + q) * B + b)
    perm = jnp.zeros((len(rows), T * B), jnp.float32).at[
        jnp.arange(len(rows)), jnp.array(rows)].set(1.0).astype(jnp.bfloat16)

    body = functools.partial(_gru_body, batch=B, seq_len=S, future_steps=F)
    return pl.pallas_call(
        body,
        out_shape=[jax.ShapeDtypeStruct((T * B, N), jnp.float32),
                   jax.ShapeDtypeStruct((Fp1 * B, N), jnp.float32)],
        grid_spec=pltpu.PrefetchScalarGridSpec(
            num_scalar_prefetch=0,
            grid=(1,),
            in_specs=[
                pl.BlockSpec((S * B, Hd), lambda i: (0, 0)),
                pl.BlockSpec((Hd, 3 * Hd), lambda i: (0, 0)),
                pl.BlockSpec((Hd, 3 * Hd), lambda i: (0, 0)),
                pl.BlockSpec((1, 3 * Hd), lambda i: (0, 0)),
                pl.BlockSpec((1, 3 * Hd), lambda i: (0, 0)),
                pl.BlockSpec((Hd, N), lambda i: (0, 0)),
                pl.BlockSpec((1, N), lambda i: (0, 0)),
                pl.BlockSpec(((T + Fp1) * B, T * B), lambda i: (0, 0)),
            ],
            out_specs=[pl.BlockSpec((T * B, N), lambda i: (0, 0)),
                       pl.BlockSpec((Fp1 * B, N), lambda i: (0, 0))],
            scratch_shapes=[pltpu.VMEM((T * B, Hd), jnp.float32)]),
        compiler_params=pltpu.CompilerParams(
            dimension_semantics=("arbitrary",)),
    )(feats_tb, w_ih, w_hh, b_ih, b_hh, w_fc, b_fc, perm)


@functools.partial(jax.jit, static_argnames=("output_dim", "height", "width",
                                             "current_index", "future_steps"))
def _forward(x, w1, b1, w2, b2, w_ih, w_hh, b_ih, b_hh, w_fc, b_fc, *,
             output_dim, height, width, current_index, future_steps):
    B, S, C, H, W = x.shape
    Hd = w_hh.shape[0]
    BS = B * S
    T = S + future_steps
    N = output_dim * height * width

    x_flat = x.reshape(BS, C, H * W)
    feat_out = _extract_features(x_flat, w1, b1, w2, b2, H, W,
                                 images_per_block=S)
    feats_tb = feat_out.reshape(S * B, Hd)

    out1, out2 = _gru_fc(feats_tb, w_ih, w_hh, b_ih, b_hh, w_fc, b_fc,
                         batch=B, seq_len=S, future_steps=future_steps,
                         current_index=current_index)

    total_output = out1.reshape(B, T, output_dim, height, width)
    future_bev = out2.reshape(B, future_steps + 1, output_dim, height, width)
    return total_output, future_bev


def kernel(x, w1, b1, w2, b2, w_ih, w_hh, b_ih, b_hh, w_fc, b_fc):
    return _forward(x, w1, b1, w2, b2, w_ih, w_hh, b_ih, b_hh, w_fc, b_fc,
                    output_dim=2, height=32, width=32,
                    current_index=2, future_steps=2)


# speedup vs baseline: 1.2113x; 1.2113x over previous
import functools

import jax
import jax.numpy as jnp
from jax.experimental import pallas as pl
from jax.experimental.pallas import tpu as pltpu


def _feat_body(x_ref, w1_ref, b1_ref, w2_ref, b2_ref, mask_ref, pool_ref,
               out_ref, xsrc_ref, stk_ref, y1s_ref, *, wimg):
    nb, cin, P0 = x_ref.shape
    cmid = w1_ref.shape[0]
    L = stk_ref.shape[1]
    E = (xsrc_ref.shape[1] - L) // 2
    shifts = [dy * wimg + dx for dy in (-1, 0, 1) for dx in (-1, 0, 1)]

    xsrc_ref[:, 0:E] = jnp.zeros((cin, E), jnp.bfloat16)
    xsrc_ref[:, E + L:] = jnp.zeros((cin, E), jnp.bfloat16)
    for k in range(nb):
        xsrc_ref[:, E + k * P0:E + (k + 1) * P0] = x_ref[k].astype(
            jnp.bfloat16)

    for t, s in enumerate(shifts):
        stk_ref[t * cin:(t + 1) * cin, :] = (
            xsrc_ref[:, E + s:E + s + L] * mask_ref[t:t + 1, :])
    acc1 = jnp.dot(w1_ref[...], stk_ref[...],
                   preferred_element_type=jnp.float32)
    y1 = jnp.maximum(acc1 + b1_ref[...], 0.0).astype(jnp.bfloat16)

    y1s_ref[:, 0:E] = jnp.zeros((cmid, E), jnp.bfloat16)
    y1s_ref[:, E + L:] = jnp.zeros((cmid, E), jnp.bfloat16)
    y1s_ref[:, E:E + L] = y1
    for t, s in enumerate(shifts):
        stk_ref[t * cmid:(t + 1) * cmid, :] = (
            y1s_ref[:, E + s:E + s + L] * mask_ref[t:t + 1, :])
    acc2 = jnp.dot(w2_ref[...], stk_ref[...],
                   preferred_element_type=jnp.float32)
    y2 = jnp.maximum(acc2 + b2_ref[...], 0.0).astype(jnp.bfloat16)

    feats = jnp.dot(y2, pool_ref[...],
                    preferred_element_type=jnp.float32)
    out_ref[...] = feats.T


def _extract_features(x_flat, w1, b1, w2, b2, H, W, images_per_block):
    BS, Cin, P0 = x_flat.shape
    Cmid = w1.shape[0]
    Chid = w2.shape[0]
    nb = images_per_block
    nblk = BS // nb
    L = nb * P0
    E = W + 1

    rr = jnp.arange(H)
    cc = jnp.arange(W)
    tap_masks = []
    for dy in (-1, 0, 1):
        for dx in (-1, 0, 1):
            rv = (rr + dy >= 0) & (rr + dy < H)
            cv = (cc + dx >= 0) & (cc + dx < W)
            tap_masks.append(jnp.tile((rv[:, None] & cv[None, :])
                                      .reshape(P0), (nb,)))
    masks = jnp.stack(tap_masks).astype(jnp.bfloat16)
    pool = jnp.kron(jnp.eye(nb, dtype=jnp.float32),
                    jnp.full((P0, 1), 1.0 / P0)).astype(jnp.bfloat16)

    body = functools.partial(_feat_body, wimg=W)
    return pl.pallas_call(
        body,
        out_shape=jax.ShapeDtypeStruct((nb, nblk * Chid), jnp.float32),
        grid_spec=pltpu.PrefetchScalarGridSpec(
            num_scalar_prefetch=0,
            grid=(nblk,),
            in_specs=[
                pl.BlockSpec((nb, Cin, P0), lambda j: (j, 0, 0)),
                pl.BlockSpec((Cmid, 9 * Cin), lambda j: (0, 0)),
                pl.BlockSpec((Cmid, 1), lambda j: (0, 0)),
                pl.BlockSpec((Chid, 9 * Cmid), lambda j: (0, 0)),
                pl.BlockSpec((Chid, 1), lambda j: (0, 0)),
                pl.BlockSpec((9, L), lambda j: (0, 0)),
                pl.BlockSpec((L, nb), lambda j: (0, 0)),
            ],
            out_specs=pl.BlockSpec((nb, Chid), lambda j: (0, j)),
            scratch_shapes=[
                pltpu.VMEM((Cin, 2 * E + L), jnp.bfloat16),
                pltpu.VMEM((9 * Cin, L), jnp.bfloat16),
                pltpu.VMEM((Cmid, 2 * E + L), jnp.bfloat16),
            ]),
        compiler_params=pltpu.CompilerParams(
            dimension_semantics=("arbitrary",)),
    )(x_flat, w1, b1, w2, b2, masks, pool)


def _gru_body(feats_ref, wih_ref, whh_ref, bih_ref, bhh_ref, wfc_ref, bfc_ref,
              perm_ref, out1_ref, out2_ref, hs_ref, *,
              batch, seq_len, future_steps):
    Hd = whh_ref.shape[0]
    B, S, F = batch, seq_len, future_steps
    n1 = out1_ref.shape[0]

    wih = wih_ref[...]
    whh = whh_ref[...]
    bih = bih_ref[...]
    bhh = bhh_ref[...]

    gi_all = jnp.dot(feats_ref[...].astype(jnp.bfloat16), wih,
                     preferred_element_type=jnp.float32) + bih

    def cell(gi, gh, h_prev):
        r = jax.nn.sigmoid(gi[:, :Hd] + gh[:, :Hd])
        z = jax.nn.sigmoid(gi[:, Hd:2 * Hd] + gh[:, Hd:2 * Hd])
        n = jnp.tanh(gi[:, 2 * Hd:] + r * gh[:, 2 * Hd:])
        return (1.0 - z) * n + z * h_prev

    h = jnp.zeros((B, Hd), jnp.float32)
    for t in range(S):
        gh = jnp.dot(h.astype(jnp.bfloat16), whh,
                     preferred_element_type=jnp.float32) + bhh
        h = cell(gi_all[t * B:(t + 1) * B, :], gh, h)
        hs_ref[t * B:(t + 1) * B, :] = h

    zero_h = jnp.zeros((B, Hd), jnp.float32)
    lh = h
    for j in range(F):
        gi = jnp.dot(lh.astype(jnp.bfloat16), wih,
                     preferred_element_type=jnp.float32) + bih
        lh = cell(gi, bhh, zero_h)
        hs_ref[(S + j) * B:(S + j + 1) * B, :] = lh

    hsp = jnp.dot(perm_ref[...], hs_ref[...].astype(jnp.bfloat16),
                  preferred_element_type=jnp.float32)
    fc = (jnp.dot(hsp.astype(jnp.bfloat16), wfc_ref[...],
                  preferred_element_type=jnp.float32) + bfc_ref[...])
    out1_ref[...] = fc[:n1]
    out2_ref[...] = fc[n1:]


def _gru_fc(feats_tb, w_ih, w_hh, b_ih, b_hh, w_fc, b_fc, *,
            batch, seq_len, future_steps, current_index):
    Hd = w_hh.shape[0]
    N = w_fc.shape[1]
    B, S, F = batch, seq_len, future_steps
    T = S + F
    Fp1 = F + 1

    rows = []
    for b in range(B):
        for t in range(T):
            rows.append(t * B + b)
    for b in range(B):
        for q in range(Fp1):
            rows.append((current_index + q) * B + b)
    perm = jnp.zeros((len(rows), T * B), jnp.float32).at[
        jnp.arange(len(rows)), jnp.array(rows)].set(1.0).astype(jnp.bfloat16)

    body = functools.partial(_gru_body, batch=B, seq_len=S, future_steps=F)
    return pl.pallas_call(
        body,
        out_shape=[jax.ShapeDtypeStruct((T * B, N), jnp.float32),
                   jax.ShapeDtypeStruct((Fp1 * B, N), jnp.float32)],
        grid_spec=pltpu.PrefetchScalarGridSpec(
            num_scalar_prefetch=0,
            grid=(1,),
            in_specs=[
                pl.BlockSpec((S * B, Hd), lambda i: (0, 0)),
                pl.BlockSpec((Hd, 3 * Hd), lambda i: (0, 0)),
                pl.BlockSpec((Hd, 3 * Hd), lambda i: (0, 0)),
                pl.BlockSpec((1, 3 * Hd), lambda i: (0, 0)),
                pl.BlockSpec((1, 3 * Hd), lambda i: (0, 0)),
                pl.BlockSpec((Hd, N), lambda i: (0, 0)),
                pl.BlockSpec((1, N), lambda i: (0, 0)),
                pl.BlockSpec(((T + Fp1) * B, T * B), lambda i: (0, 0)),
            ],
            out_specs=[pl.BlockSpec((T * B, N), lambda i: (0, 0)),
                       pl.BlockSpec((Fp1 * B, N), lambda i: (0, 0))],
            scratch_shapes=[pltpu.VMEM((T * B, Hd), jnp.float32)]),
        compiler_params=pltpu.CompilerParams(
            dimension_semantics=("arbitrary",)),
    )(feats_tb, w_ih, w_hh, b_ih, b_hh, w_fc, b_fc, perm)


@functools.partial(jax.jit, static_argnames=("output_dim", "height", "width",
                                             "current_index", "future_steps"))
def _forward(x, w1, b1, w2, b2, w_ih, w_hh, b_ih, b_hh, w_fc, b_fc, *,
             output_dim, height, width, current_index, future_steps):
    B, S, C, H, W = x.shape
    Hd = w_hh.shape[0]
    BS = B * S
    T = S + future_steps
    N = output_dim * height * width

    x_flat = x.reshape(BS, C, H * W)
    feat_out = _extract_features(x_flat, w1, b1, w2, b2, H, W,
                                 images_per_block=S)
    feats_tb = feat_out.reshape(S * B, Hd)

    out1, out2 = _gru_fc(feats_tb, w_ih, w_hh, b_ih, b_hh, w_fc, b_fc,
                         batch=B, seq_len=S, future_steps=future_steps,
                         current_index=current_index)

    total_output = out1.reshape(B, T, output_dim, height, width)
    future_bev = out2.reshape(B, future_steps + 1, output_dim, height, width)
    return total_output, future_bev


def kernel(x, w1, b1, w2, b2, w_ih, w_hh, b_ih, b_hh, w_fc, b_fc):
    return _forward(x, w1, b1, w2, b2, w_ih, w_hh, b_ih, b_hh, w_fc, b_fc,
                    output_dim=2, height=32, width=32,
                    current_index=2, future_steps=2)


# speedup vs baseline: 1.2682x; 1.0470x over previous
import functools

import jax
import jax.numpy as jnp
from jax.experimental import pallas as pl
from jax.experimental.pallas import tpu as pltpu


def _feat_body(x_ref, w1_ref, b1_ref, w2_ref, b2_ref, mask_ref, pool_ref,
               out_ref, xsrc_ref, stk_ref, y1s_ref, *, wimg):
    nb, cin, P0 = x_ref.shape
    cmid = w1_ref.shape[0]
    L = stk_ref.shape[1]
    E = (xsrc_ref.shape[1] - L) // 2
    shifts = [dy * wimg + dx for dy in (-1, 0, 1) for dx in (-1, 0, 1)]

    xsrc_ref[:, 0:E] = jnp.zeros((cin, E), jnp.bfloat16)
    xsrc_ref[:, E + L:] = jnp.zeros((cin, E), jnp.bfloat16)
    for k in range(nb):
        xsrc_ref[:, E + k * P0:E + (k + 1) * P0] = x_ref[k].astype(
            jnp.bfloat16)

    for t, s in enumerate(shifts):
        src = xsrc_ref[:, E + s:E + s + L]
        stk_ref[t * cin:(t + 1) * cin, :] = (
            src if s == 0 else src * mask_ref[t:t + 1, :])
    acc1 = jnp.dot(w1_ref[...], stk_ref[...],
                   preferred_element_type=jnp.float32)
    y1 = jnp.maximum(acc1 + b1_ref[...], 0.0).astype(jnp.bfloat16)

    y1s_ref[:, 0:E] = jnp.zeros((cmid, E), jnp.bfloat16)
    y1s_ref[:, E + L:] = jnp.zeros((cmid, E), jnp.bfloat16)
    y1s_ref[:, E:E + L] = y1
    for t, s in enumerate(shifts):
        src = y1s_ref[:, E + s:E + s + L]
        stk_ref[t * cmid:(t + 1) * cmid, :] = (
            src if s == 0 else src * mask_ref[t:t + 1, :])
    acc2 = jnp.dot(w2_ref[...], stk_ref[...],
                   preferred_element_type=jnp.float32)
    y2 = jnp.maximum(acc2 + b2_ref[...], 0.0).astype(jnp.bfloat16)

    feats = jnp.dot(y2, pool_ref[...],
                    preferred_element_type=jnp.float32)
    out_ref[...] = feats.T


def _extract_features(x_flat, w1, b1, w2, b2, H, W, images_per_block):
    BS, Cin, P0 = x_flat.shape
    Cmid = w1.shape[0]
    Chid = w2.shape[0]
    nb = images_per_block
    nblk = BS // nb
    L = nb * P0
    E = W + 1

    rr = jnp.arange(H)
    cc = jnp.arange(W)
    tap_masks = []
    for dy in (-1, 0, 1):
        for dx in (-1, 0, 1):
            rv = (rr + dy >= 0) & (rr + dy < H)
            cv = (cc + dx >= 0) & (cc + dx < W)
            tap_masks.append(jnp.tile((rv[:, None] & cv[None, :])
                                      .reshape(P0), (nb,)))
    masks = jnp.stack(tap_masks).astype(jnp.bfloat16)
    pool = jnp.kron(jnp.eye(nb, dtype=jnp.float32),
                    jnp.full((P0, 1), 1.0 / P0)).astype(jnp.bfloat16)

    body = functools.partial(_feat_body, wimg=W)
    return pl.pallas_call(
        body,
        out_shape=jax.ShapeDtypeStruct((nb, nblk * Chid), jnp.float32),
        grid_spec=pltpu.PrefetchScalarGridSpec(
            num_scalar_prefetch=0,
            grid=(nblk,),
            in_specs=[
                pl.BlockSpec((nb, Cin, P0), lambda j: (j, 0, 0)),
                pl.BlockSpec((Cmid, 9 * Cin), lambda j: (0, 0)),
                pl.BlockSpec((Cmid, 1), lambda j: (0, 0)),
                pl.BlockSpec((Chid, 9 * Cmid), lambda j: (0, 0)),
                pl.BlockSpec((Chid, 1), lambda j: (0, 0)),
                pl.BlockSpec((9, L), lambda j: (0, 0)),
                pl.BlockSpec((L, nb), lambda j: (0, 0)),
            ],
            out_specs=pl.BlockSpec((nb, Chid), lambda j: (0, j)),
            scratch_shapes=[
                pltpu.VMEM((Cin, 2 * E + L), jnp.bfloat16),
                pltpu.VMEM((9 * Cin, L), jnp.bfloat16),
                pltpu.VMEM((Cmid, 2 * E + L), jnp.bfloat16),
            ]),
        compiler_params=pltpu.CompilerParams(
            dimension_semantics=("arbitrary",)),
    )(x_flat, w1, b1, w2, b2, masks, pool)


def _gru_body(feats_ref, wih_ref, whh_ref, bih_ref, bhh_ref, wfc_ref, bfc_ref,
              permf_ref, perm_ref, out1_ref, out2_ref, hs_ref, *,
              batch, seq_len, future_steps):
    Hd = whh_ref.shape[0]
    B, S, F = batch, seq_len, future_steps
    n1 = out1_ref.shape[0]

    wih = wih_ref[...]
    whh = whh_ref[...]
    bih = bih_ref[...]
    bhh = bhh_ref[...]

    fp = jnp.dot(permf_ref[...], feats_ref[...].astype(jnp.bfloat16),
                 preferred_element_type=jnp.float32)
    gi_all = jnp.dot(fp.astype(jnp.bfloat16), wih,
                     preferred_element_type=jnp.float32) + bih

    def cell(gi, gh, h_prev):
        r = jax.nn.sigmoid(gi[:, :Hd] + gh[:, :Hd])
        z = jax.nn.sigmoid(gi[:, Hd:2 * Hd] + gh[:, Hd:2 * Hd])
        n = jnp.tanh(gi[:, 2 * Hd:] + r * gh[:, 2 * Hd:])
        return (1.0 - z) * n + z * h_prev

    h = jnp.zeros((B, Hd), jnp.float32)
    for t in range(S):
        gh = jnp.dot(h.astype(jnp.bfloat16), whh,
                     preferred_element_type=jnp.float32) + bhh
        h = cell(gi_all[t * B:(t + 1) * B, :], gh, h)
        hs_ref[t * B:(t + 1) * B, :] = h

    zero_h = jnp.zeros((B, Hd), jnp.float32)
    lh = h
    for j in range(F):
        gi = jnp.dot(lh.astype(jnp.bfloat16), wih,
                     preferred_element_type=jnp.float32) + bih
        lh = cell(gi, bhh, zero_h)
        hs_ref[(S + j) * B:(S + j + 1) * B, :] = lh

    hsp = jnp.dot(perm_ref[...], hs_ref[...].astype(jnp.bfloat16),
                  preferred_element_type=jnp.float32)
    fc = (jnp.dot(hsp.astype(jnp.bfloat16), wfc_ref[...],
                  preferred_element_type=jnp.float32) + bfc_ref[...])
    out1_ref[...] = fc[:n1]
    out2_ref[...] = fc[n1:]


def _gru_fc(feats_nat, w_ih, w_hh, b_ih, b_hh, w_fc, b_fc, *,
            batch, seq_len, future_steps, current_index, images_per_block):
    Hd = w_hh.shape[0]
    N = w_fc.shape[1]
    B, S, F = batch, seq_len, future_steps
    T = S + F
    Fp1 = F + 1
    nb = images_per_block
    nblk = (B * S) // nb

    frows = []
    for t in range(S):
        for b in range(B):
            i = b * S + t
            frows.append((i % nb) * nblk + i // nb)
    permf = jnp.zeros((S * B, S * B), jnp.float32).at[
        jnp.arange(S * B), jnp.array(frows)].set(1.0).astype(jnp.bfloat16)

    rows = []
    for b in range(B):
        for t in range(T):
            rows.append(t * B + b)
    for b in range(B):
        for q in range(Fp1):
            rows.append((current_index + q) * B + b)
    perm = jnp.zeros((len(rows), T * B), jnp.float32).at[
        jnp.arange(len(rows)), jnp.array(rows)].set(1.0).astype(jnp.bfloat16)

    body = functools.partial(_gru_body, batch=B, seq_len=S, future_steps=F)
    return pl.pallas_call(
        body,
        out_shape=[jax.ShapeDtypeStruct((T * B, N), jnp.float32),
                   jax.ShapeDtypeStruct((Fp1 * B, N), jnp.float32)],
        grid_spec=pltpu.PrefetchScalarGridSpec(
            num_scalar_prefetch=0,
            grid=(1,),
            in_specs=[
                pl.BlockSpec((S * B, Hd), lambda i: (0, 0)),
                pl.BlockSpec((Hd, 3 * Hd), lambda i: (0, 0)),
                pl.BlockSpec((Hd, 3 * Hd), lambda i: (0, 0)),
                pl.BlockSpec((1, 3 * Hd), lambda i: (0, 0)),
                pl.BlockSpec((1, 3 * Hd), lambda i: (0, 0)),
                pl.BlockSpec((Hd, N), lambda i: (0, 0)),
                pl.BlockSpec((1, N), lambda i: (0, 0)),
                pl.BlockSpec((S * B, S * B), lambda i: (0, 0)),
                pl.BlockSpec(((T + Fp1) * B, T * B), lambda i: (0, 0)),
            ],
            out_specs=[pl.BlockSpec((T * B, N), lambda i: (0, 0)),
                       pl.BlockSpec((Fp1 * B, N), lambda i: (0, 0))],
            scratch_shapes=[pltpu.VMEM((T * B, Hd), jnp.float32)]),
        compiler_params=pltpu.CompilerParams(
            dimension_semantics=("arbitrary",)),
    )(feats_nat, w_ih, w_hh, b_ih, b_hh, w_fc, b_fc, permf, perm)


@functools.partial(jax.jit, static_argnames=("output_dim", "height", "width",
                                             "current_index", "future_steps"))
def _forward(x, w1, b1, w2, b2, w_ih, w_hh, b_ih, b_hh, w_fc, b_fc, *,
             output_dim, height, width, current_index, future_steps):
    B, S, C, H, W = x.shape
    Hd = w_hh.shape[0]
    BS = B * S
    T = S + future_steps
    N = output_dim * height * width

    x_flat = x.reshape(BS, C, H * W)
    nb = 16
    feat_out = _extract_features(x_flat, w1, b1, w2, b2, H, W,
                                 images_per_block=nb)
    feats_nat = feat_out.reshape(BS, Hd)

    out1, out2 = _gru_fc(feats_nat, w_ih, w_hh, b_ih, b_hh, w_fc, b_fc,
                         batch=B, seq_len=S, future_steps=future_steps,
                         current_index=current_index, images_per_block=nb)

    total_output = out1.reshape(B, T, output_dim, height, width)
    future_bev = out2.reshape(B, future_steps + 1, output_dim, height, width)
    return total_output, future_bev


def kernel(x, w1, b1, w2, b2, w_ih, w_hh, b_ih, b_hh, w_fc, b_fc):
    return _forward(x, w1, b1, w2, b2, w_ih, w_hh, b_ih, b_hh, w_fc, b_fc,
                    output_dim=2, height=32, width=32,
                    current_index=2, future_steps=2)


# speedup vs baseline: 1.2825x; 1.0112x over previous
import functools

import jax
import jax.numpy as jnp
from jax.experimental import pallas as pl
from jax.experimental.pallas import tpu as pltpu


def _body(x_ref, w1_ref, b1_ref, w2_ref, b2_ref, mask_ref, pool_ref,
          wih_ref, whh_ref, bih_ref, bhh_ref, wfc_ref, bfc_ref,
          permf_ref, perm_ref, out1_ref, out2_ref,
          xsrc_ref, stk_ref, y1s_ref, fs_ref, hs_ref, *,
          wimg, nblk, batch, seq_len, future_steps):
    nb, cin, P0 = x_ref.shape
    cmid = w1_ref.shape[0]
    L = stk_ref.shape[1]
    E = (xsrc_ref.shape[1] - L) // 2
    shifts = [dy * wimg + dx for dy in (-1, 0, 1) for dx in (-1, 0, 1)]
    j = pl.program_id(0)

    xsrc_ref[:, 0:E] = jnp.zeros((cin, E), jnp.bfloat16)
    xsrc_ref[:, E + L:] = jnp.zeros((cin, E), jnp.bfloat16)
    for k in range(nb):
        xsrc_ref[:, E + k * P0:E + (k + 1) * P0] = x_ref[k].astype(
            jnp.bfloat16)

    for t, s in enumerate(shifts):
        src = xsrc_ref[:, E + s:E + s + L]
        stk_ref[t * cin:(t + 1) * cin, :] = (
            src if s == 0 else src * mask_ref[t:t + 1, :])
    acc1 = jnp.dot(w1_ref[...], stk_ref[...],
                   preferred_element_type=jnp.float32)
    y1 = jnp.maximum(acc1 + b1_ref[...], 0.0).astype(jnp.bfloat16)

    y1s_ref[:, 0:E] = jnp.zeros((cmid, E), jnp.bfloat16)
    y1s_ref[:, E + L:] = jnp.zeros((cmid, E), jnp.bfloat16)
    y1s_ref[:, E:E + L] = y1
    for t, s in enumerate(shifts):
        src = y1s_ref[:, E + s:E + s + L]
        stk_ref[t * cmid:(t + 1) * cmid, :] = (
            src if s == 0 else src * mask_ref[t:t + 1, :])
    acc2 = jnp.dot(w2_ref[...], stk_ref[...],
                   preferred_element_type=jnp.float32)
    y2 = jnp.maximum(acc2 + b2_ref[...], 0.0).astype(jnp.bfloat16)

    feats = jnp.dot(y2, pool_ref[...],
                    preferred_element_type=jnp.float32)
    fs_ref[pl.ds(j * nb, nb), :] = feats.T

    @pl.when(j == nblk - 1)
    def _gru():
        Hd = whh_ref.shape[0]
        B, S, F = batch, seq_len, future_steps
        n1 = out1_ref.shape[0]

        wih = wih_ref[...]
        whh = whh_ref[...]
        bih = bih_ref[...]
        bhh = bhh_ref[...]

        fp = jnp.dot(permf_ref[...], fs_ref[...].astype(jnp.bfloat16),
                     preferred_element_type=jnp.float32)
        gi_all = jnp.dot(fp.astype(jnp.bfloat16), wih,
                         preferred_element_type=jnp.float32) + bih

        def cell(gi, gh, h_prev):
            r = jax.nn.sigmoid(gi[:, :Hd] + gh[:, :Hd])
            z = jax.nn.sigmoid(gi[:, Hd:2 * Hd] + gh[:, Hd:2 * Hd])
            n = jnp.tanh(gi[:, 2 * Hd:] + r * gh[:, 2 * Hd:])
            return (1.0 - z) * n + z * h_prev

        h = jnp.zeros((B, Hd), jnp.float32)
        for t in range(S):
            gh = jnp.dot(h.astype(jnp.bfloat16), whh,
                         preferred_element_type=jnp.float32) + bhh
            h = cell(gi_all[t * B:(t + 1) * B, :], gh, h)
            hs_ref[t * B:(t + 1) * B, :] = h

        zero_h = jnp.zeros((B, Hd), jnp.float32)
        lh = h
        for q in range(F):
            gi = jnp.dot(lh.astype(jnp.bfloat16), wih,
                         preferred_element_type=jnp.float32) + bih
            lh = cell(gi, bhh, zero_h)
            hs_ref[(S + q) * B:(S + q + 1) * B, :] = lh

        hsp = jnp.dot(perm_ref[...], hs_ref[...].astype(jnp.bfloat16),
                      preferred_element_type=jnp.float32)
        fc = (jnp.dot(hsp.astype(jnp.bfloat16), wfc_ref[...],
                      preferred_element_type=jnp.float32) + bfc_ref[...])
        out1_ref[...] = fc[:n1]
        out2_ref[...] = fc[n1:]


@functools.partial(jax.jit, static_argnames=("output_dim", "height", "width",
                                             "current_index", "future_steps"))
def _forward(x, w1, b1, w2, b2, w_ih, w_hh, b_ih, b_hh, w_fc, b_fc, *,
             output_dim, height, width, current_index, future_steps):
    B, S, C, H, W = x.shape
    Hd = w_hh.shape[0]
    BS = B * S
    F = future_steps
    T = S + F
    Fp1 = F + 1
    N = output_dim * height * width
    Cmid = w1.shape[0]
    Chid = w2.shape[0]

    nb = 16
    nblk = BS // nb
    P0 = H * W
    L = nb * P0
    E = W + 1

    x_flat = x.reshape(BS, C, P0)

    rr = jnp.arange(H)
    cc = jnp.arange(W)
    tap_masks = []
    for dy in (-1, 0, 1):
        for dx in (-1, 0, 1):
            rv = (rr + dy >= 0) & (rr + dy < H)
            cv = (cc + dx >= 0) & (cc + dx < W)
            tap_masks.append(jnp.tile((rv[:, None] & cv[None, :])
                                      .reshape(P0), (nb,)))
    masks = jnp.stack(tap_masks).astype(jnp.bfloat16)
    pool = jnp.kron(jnp.eye(nb, dtype=jnp.float32),
                    jnp.full((P0, 1), 1.0 / P0)).astype(jnp.bfloat16)

    frows = [b * S + t for t in range(S) for b in range(B)]
    permf = jnp.zeros((S * B, S * B), jnp.float32).at[
        jnp.arange(S * B), jnp.array(frows)].set(1.0).astype(jnp.bfloat16)

    rows = [t * B + b for b in range(B) for t in range(T)]
    rows += [(current_index + q) * B + b for b in range(B)
             for q in range(Fp1)]
    perm = jnp.zeros((len(rows), T * B), jnp.float32).at[
        jnp.arange(len(rows)), jnp.array(rows)].set(1.0).astype(jnp.bfloat16)

    body = functools.partial(_body, wimg=W, nblk=nblk, batch=B, seq_len=S,
                             future_steps=F)
    out1, out2 = pl.pallas_call(
        body,
        out_shape=[jax.ShapeDtypeStruct((T * B, N), jnp.float32),
                   jax.ShapeDtypeStruct((Fp1 * B, N), jnp.float32)],
        grid_spec=pltpu.PrefetchScalarGridSpec(
            num_scalar_prefetch=0,
            grid=(nblk,),
            in_specs=[
                pl.BlockSpec((nb, C, P0), lambda j: (j, 0, 0)),
                pl.BlockSpec((Cmid, 9 * C), lambda j: (0, 0)),
                pl.BlockSpec((Cmid, 1), lambda j: (0, 0)),
                pl.BlockSpec((Chid, 9 * Cmid), lambda j: (0, 0)),
                pl.BlockSpec((Chid, 1), lambda j: (0, 0)),
                pl.BlockSpec((9, L), lambda j: (0, 0)),
                pl.BlockSpec((L, nb), lambda j: (0, 0)),
                pl.BlockSpec((Hd, 3 * Hd), lambda j: (0, 0)),
                pl.BlockSpec((Hd, 3 * Hd), lambda j: (0, 0)),
                pl.BlockSpec((1, 3 * Hd), lambda j: (0, 0)),
                pl.BlockSpec((1, 3 * Hd), lambda j: (0, 0)),
                pl.BlockSpec((Hd, N), lambda j: (0, 0)),
                pl.BlockSpec((1, N), lambda j: (0, 0)),
                pl.BlockSpec((S * B, S * B), lambda j: (0, 0)),
                pl.BlockSpec(((T + Fp1) * B, T * B), lambda j: (0, 0)),
            ],
            out_specs=[pl.BlockSpec((T * B, N), lambda j: (0, 0)),
                       pl.BlockSpec((Fp1 * B, N), lambda j: (0, 0))],
            scratch_shapes=[
                pltpu.VMEM((C, 2 * E + L), jnp.bfloat16),
                pltpu.VMEM((9 * C, L), jnp.bfloat16),
                pltpu.VMEM((Cmid, 2 * E + L), jnp.bfloat16),
                pltpu.VMEM((S * B, Chid), jnp.float32),
                pltpu.VMEM((T * B, Hd), jnp.float32),
            ]),
        compiler_params=pltpu.CompilerParams(
            dimension_semantics=("arbitrary",)),
    )(x_flat, w1, b1, w2, b2, masks, pool,
      w_ih, w_hh, b_ih, b_hh, w_fc, b_fc, permf, perm)

    total_output = out1.reshape(B, T, output_dim, height, width)
    future_bev = out2.reshape(B, Fp1, output_dim, height, width)
    return total_output, future_bev


def kernel(x, w1, b1, w2, b2, w_ih, w_hh, b_ih, b_hh, w_fc, b_fc):
    return _forward(x, w1, b1, w2, b2, w_ih, w_hh, b_ih, b_hh, w_fc, b_fc,
                    output_dim=2, height=32, width=32,
                    current_index=2, future_steps=2)
